# baseline (speedup 1.0000x reference)
import functools

import jax
import jax.numpy as jnp
from jax import lax
from jax.experimental import pallas as pl
from jax.experimental.pallas import tpu as pltpu
from jax.experimental.pallas import tpu_sc as plsc

_NC = 2
_NS = 16
_NW = _NC * _NS

_C = 128
_NBUF = 4
_W = 128


def _make_sc_gather(nch: int, d: int):
    mesh = plsc.VectorSubcoreMesh(core_axis_name="c", subcore_axis_name="s")
    b_total = _NW * nch * _C

    @functools.partial(
        pl.kernel,
        mesh=mesh,
        out_type=jax.ShapeDtypeStruct((b_total, _W), jnp.float32),
        scratch_types=(
            [
                pltpu.VMEM((nch, _C), jnp.int32),
                pltpu.VMEM((_NBUF, _C, _W), jnp.float32),
            ]
            + [pltpu.SemaphoreType.DMA] * _NBUF
            + [pltpu.SemaphoreType.DMA] * _NBUF
        ),
    )
    def sc_gather(ids_hbm, table_hbm, out_hbm, idx_v, rows_v, *sems):
        gsems = sems[:_NBUF]
        ssems = sems[_NBUF:]
        wid = lax.axis_index("s") * _NC + lax.axis_index("c")
        base = wid * (nch * _C)
        pltpu.sync_copy(ids_hbm.at[wid], idx_v)

        def group(g, carry):
            gds = []
            for b in range(_NBUF):
                i = g * _NBUF + b
                gds.append(
                    pltpu.async_copy(
                        table_hbm.at[idx_v.at[i]], rows_v.at[b], gsems[b]
                    )
                )
            sds = []
            for b in range(_NBUF):
                i = g * _NBUF + b
                gds[b].wait()
                sds.append(
                    pltpu.async_copy(
                        rows_v.at[b],
                        out_hbm.at[pl.ds(base + i * _C, _C)],
                        ssems[b],
                    )
                )
            for b in range(_NBUF):
                sds[b].wait()
            return carry

        lax.fori_loop(0, nch // _NBUF, group, 0)

    return sc_gather


def kernel(input, table):
    bsz, hist = input.shape
    d = table.shape[1]
    n = bsz * hist
    assert n % (_NW * _C * _NBUF) == 0
    nch = n // (_NW * _C)
    ids = input.T.astype(jnp.int32).reshape(_NW, nch, _C)
    table_pad = jnp.pad(table, ((0, 0), (0, _W - d)))
    rows = _make_sc_gather(nch, d)(ids, table_pad)
    return jnp.transpose(rows.reshape(hist, bsz, _W)[:, :, :d], (1, 0, 2))


# speedup vs baseline: 1.1115x; 1.1115x over previous
import functools

import jax
import jax.numpy as jnp
from jax import lax
from jax.experimental import pallas as pl
from jax.experimental.pallas import tpu as pltpu
from jax.experimental.pallas import tpu_sc as plsc

_NC = 2
_NS = 16
_NW = _NC * _NS

_C = 128
_NBUF = 8
_W = 128


def _make_sc_gather(nch: int, d: int):
    mesh = plsc.VectorSubcoreMesh(core_axis_name="c", subcore_axis_name="s")
    b_total = _NW * nch * _C

    @functools.partial(
        pl.kernel,
        mesh=mesh,
        out_type=jax.ShapeDtypeStruct((b_total, _W), jnp.float32),
        scratch_types=(
            [
                pltpu.VMEM((nch, _C), jnp.int32),
                pltpu.VMEM((_NBUF, _C, d), jnp.float32),
            ]
            + [pltpu.SemaphoreType.DMA] * _NBUF
            + [pltpu.SemaphoreType.DMA] * _NBUF
        ),
        compiler_params=pltpu.CompilerParams(use_tc_tiling_on_sc=False),
    )
    def sc_gather(ids_hbm, table_hbm, out_hbm, idx_v, rows_v, *sems):
        gsems = sems[:_NBUF]
        ssems = sems[_NBUF:]
        wid = lax.axis_index("s") * _NC + lax.axis_index("c")
        base = wid * (nch * _C)
        pltpu.sync_copy(ids_hbm.at[wid], idx_v)

        def group(g, carry):
            gds = []
            for b in range(_NBUF):
                i = g * _NBUF + b
                gds.append(
                    pltpu.async_copy(
                        table_hbm.at[idx_v.at[i]], rows_v.at[b], gsems[b]
                    )
                )
            sds = []
            for b in range(_NBUF):
                i = g * _NBUF + b
                gds[b].wait()
                sds.append(
                    pltpu.async_copy(
                        rows_v.at[b],
                        out_hbm.at[pl.ds(base + i * _C, _C), pl.ds(0, d)],
                        ssems[b],
                    )
                )
            for b in range(_NBUF):
                sds[b].wait()
            return carry

        lax.fori_loop(0, nch // _NBUF, group, 0)

    return sc_gather


def kernel(input, table):
    bsz, hist = input.shape
    d = table.shape[1]
    n = bsz * hist
    assert n % (_NW * _C * _NBUF) == 0
    nch = n // (_NW * _C)
    ids = input.T.astype(jnp.int32).reshape(_NW, nch, _C)
    rows = _make_sc_gather(nch, d)(ids, table)
    return jnp.transpose(rows.reshape(hist, bsz, _W)[:, :, :d], (1, 0, 2))
